# baseline (speedup 1.0000x reference)
import functools

import jax
import jax.numpy as jnp
from jax import lax
from jax.experimental import pallas as pl
from jax.experimental.pallas import tpu as pltpu
from jax.experimental.pallas import tpu_sc as plsc

NC = 2
NS = 16
NW = NC * NS

CHUNK = 128
ZROWS = 32


def _segment_sum_sc(edge_state, edge_dst, n_pad):
    n_edges, d = edge_state.shape
    nload = n_edges // (NW * CHUNK)
    assert nload % 2 == 0 and nload >= 4
    npair = nload // 2
    epw = nload * CHUNK
    ntail = n_edges - NW * epw
    assert ntail % CHUNK == 0
    tail_chunks = ntail // CHUNK
    assert tail_chunks <= NW
    rows_per_tile = n_pad // NS
    assert rows_per_tile * NS == n_pad and rows_per_tile % 8 == 0
    assert rows_per_tile % ZROWS == 0
    nzcopy = rows_per_tile // ZROWS

    mesh = plsc.VectorSubcoreMesh(core_axis_name="c", subcore_axis_name="s")

    @functools.partial(
        pl.kernel,
        out_type=jax.ShapeDtypeStruct((NC, n_pad, d), jnp.float32),
        mesh=mesh,
        scratch_types=[
            pltpu.VMEM_SHARED((n_pad, d), jnp.float32),
            pltpu.VMEM((CHUNK, d), jnp.float32),
            pltpu.VMEM((CHUNK, d), jnp.float32),
            pltpu.VMEM((CHUNK,), jnp.int32),
            pltpu.VMEM((CHUNK,), jnp.int32),
            pltpu.VMEM((ZROWS, d), jnp.float32),
            pltpu.SemaphoreType.DMA,
            pltpu.SemaphoreType.DMA,
            pltpu.SemaphoreType.DMA,
            pltpu.SemaphoreType.DMA,
            pltpu.SemaphoreType.DMA,
            pltpu.SemaphoreType.DMA,
        ],
    )
    def seg_sum(es_hbm, dst_hbm, out_hbm, acc, rows_a, rows_b, idx_a, idx_b,
                zbuf, sem_ra, sem_rb, sem_ia, sem_ib, sem_sa, sem_sb):
        cid = lax.axis_index("c")
        sid = lax.axis_index("s")
        wid = sid * NC + cid

        zero16 = jnp.zeros((16,), jnp.float32)
        lanes = d // 16

        def zstore(i, carry):
            zbuf[i // lanes, pl.ds((i % lanes) * 16, 16)] = zero16
            return carry

        lax.fori_loop(0, ZROWS * lanes, zstore, 0)

        def zcopy(k, carry):
            pltpu.sync_copy(zbuf, acc.at[pl.ds(sid * rows_per_tile + k * ZROWS, ZROWS)])
            return carry

        lax.fori_loop(0, nzcopy, zcopy, 0)
        plsc.subcore_barrier()

        def start_load(li, rbuf, ibuf, rsem, isem):
            base = wid * epw + li * CHUNK
            pltpu.async_copy(es_hbm.at[pl.ds(base, CHUNK)], rbuf, rsem)
            pltpu.async_copy(dst_hbm.at[pl.ds(base, CHUNK)], ibuf, isem)

        def wait_load(rbuf, ibuf, rsem, isem):
            pltpu.make_async_copy(es_hbm.at[pl.ds(0, CHUNK)], rbuf, rsem).wait()
            pltpu.make_async_copy(dst_hbm.at[pl.ds(0, CHUNK)], ibuf, isem).wait()

        def start_scatter(rbuf, ibuf, sem):
            pltpu.async_copy(rbuf, acc.at[ibuf], sem, add=True)

        def wait_scatter(rbuf, ibuf, sem):
            pltpu.make_async_copy(rbuf, acc.at[ibuf], sem).wait()

        start_load(0, rows_a, idx_a, sem_ra, sem_ia)
        start_load(1, rows_b, idx_b, sem_rb, sem_ib)

        def body(j, carry):
            wait_load(rows_a, idx_a, sem_ra, sem_ia)
            start_scatter(rows_a, idx_a, sem_sa)
            wait_load(rows_b, idx_b, sem_rb, sem_ib)
            start_scatter(rows_b, idx_b, sem_sb)
            wait_scatter(rows_a, idx_a, sem_sa)
            start_load(2 * j + 2, rows_a, idx_a, sem_ra, sem_ia)
            wait_scatter(rows_b, idx_b, sem_sb)
            start_load(2 * j + 3, rows_b, idx_b, sem_rb, sem_ib)
            return carry

        lax.fori_loop(0, npair - 1, body, 0)
        wait_load(rows_a, idx_a, sem_ra, sem_ia)
        start_scatter(rows_a, idx_a, sem_sa)
        wait_load(rows_b, idx_b, sem_rb, sem_ib)
        start_scatter(rows_b, idx_b, sem_sb)
        wait_scatter(rows_a, idx_a, sem_sa)
        wait_scatter(rows_b, idx_b, sem_sb)

        if tail_chunks:
            @pl.when(wid < tail_chunks)
            def _tail():
                base = NW * epw + wid * CHUNK
                pltpu.sync_copy(es_hbm.at[pl.ds(base, CHUNK)], rows_a)
                pltpu.sync_copy(dst_hbm.at[pl.ds(base, CHUNK)], idx_a)
                pltpu.sync_copy(rows_a, acc.at[idx_a], add=True)

        plsc.subcore_barrier()

        pltpu.sync_copy(
            acc.at[pl.ds(sid * rows_per_tile, rows_per_tile)],
            out_hbm.at[cid, pl.ds(sid * rows_per_tile, rows_per_tile)],
        )

    return seg_sum(edge_state, edge_dst)


def _dense_body(nf_ref, p_ref, w_ref, b_ref, o_ref):
    d = nf_ref.shape[1]
    ns = p_ref[0] + p_ref[1]
    x = jnp.dot(nf_ref[...], w_ref[0:d, :], preferred_element_type=jnp.float32)
    y = jnp.dot(ns, w_ref[d:, :], preferred_element_type=jnp.float32)
    o_ref[...] = jnp.maximum(x + y + b_ref[...], 0.0)


def kernel(node_feature, edge_state, edge_dst, W, b):
    n_nodes, d = node_feature.shape
    units = W.shape[1]
    g = NS * ZROWS
    n_pad = ((n_nodes + g - 1) // g) * g
    partials = _segment_sum_sc(edge_state, edge_dst.astype(jnp.int32), n_pad)

    blk = 2000
    assert n_nodes % blk == 0
    grid = (n_nodes // blk,)
    out = pl.pallas_call(
        _dense_body,
        grid=grid,
        in_specs=[
            pl.BlockSpec((blk, d), lambda i: (i, 0)),
            pl.BlockSpec((NC, blk, d), lambda i: (0, i, 0)),
            pl.BlockSpec(W.shape, lambda i: (0, 0)),
            pl.BlockSpec((1, units), lambda i: (0, 0)),
        ],
        out_specs=pl.BlockSpec((blk, units), lambda i: (i, 0)),
        out_shape=jax.ShapeDtypeStruct((n_nodes, units), jnp.float32),
    )(node_feature, partials, W, b.reshape(1, units))
    return out


# speedup vs baseline: 1.1511x; 1.1511x over previous
import functools

import jax
import jax.numpy as jnp
from jax import lax
from jax.experimental import pallas as pl
from jax.experimental.pallas import tpu as pltpu
from jax.experimental.pallas import tpu_sc as plsc

NC = 2
NS = 16
NW = NC * NS

CHUNK = 128
ZROWS = 32


def _segment_sum_sc(edge_state, edge_dst, n_nodes, n_pad):
    n_edges, d = edge_state.shape
    nload = n_edges // (NW * CHUNK)
    assert nload % 2 == 0 and nload >= 4
    npair = nload // 2
    epw = nload * CHUNK
    ntail = n_edges - NW * epw
    assert ntail % CHUNK == 0
    tail_chunks = ntail // CHUNK
    assert tail_chunks <= NW
    rows_per_tile = n_pad // NS
    assert rows_per_tile * NS == n_pad and rows_per_tile % 8 == 0
    assert rows_per_tile % ZROWS == 0
    nzcopy = rows_per_tile // ZROWS
    last_rows = n_nodes - (NS - 1) * rows_per_tile
    assert 0 < last_rows <= rows_per_tile and last_rows % 8 == 0

    mesh = plsc.VectorSubcoreMesh(core_axis_name="c", subcore_axis_name="s")

    @functools.partial(
        pl.kernel,
        out_type=jax.ShapeDtypeStruct((NC, n_nodes, d), jnp.float32),
        mesh=mesh,
        scratch_types=[
            pltpu.VMEM_SHARED((n_pad, d), jnp.float32),
            pltpu.VMEM((CHUNK, d), jnp.float32),
            pltpu.VMEM((CHUNK, d), jnp.float32),
            pltpu.VMEM((CHUNK,), jnp.int32),
            pltpu.VMEM((CHUNK,), jnp.int32),
            pltpu.VMEM((ZROWS, d), jnp.float32),
            pltpu.SemaphoreType.DMA,
            pltpu.SemaphoreType.DMA,
            pltpu.SemaphoreType.DMA,
            pltpu.SemaphoreType.DMA,
        ],
    )
    def seg_sum(es_hbm, dst_hbm, out_hbm, acc, rows_a, rows_b, idx_a, idx_b,
                zbuf, sem_ra, sem_rb, sem_ia, sem_ib):
        cid = lax.axis_index("c")
        sid = lax.axis_index("s")
        wid = sid * NC + cid

        def start_load(li, rbuf, ibuf, rsem, isem):
            base = wid * epw + li * CHUNK
            pltpu.async_copy(es_hbm.at[pl.ds(base, CHUNK)], rbuf, rsem)
            pltpu.async_copy(dst_hbm.at[pl.ds(base, CHUNK)], ibuf, isem)

        def wait_load(rbuf, ibuf, rsem, isem):
            pltpu.make_async_copy(es_hbm.at[pl.ds(0, CHUNK)], rbuf, rsem).wait()
            pltpu.make_async_copy(dst_hbm.at[pl.ds(0, CHUNK)], ibuf, isem).wait()

        start_load(0, rows_a, idx_a, sem_ra, sem_ia)

        zero16 = jnp.zeros((16,), jnp.float32)
        lanes = d // 16

        def zstore(i, carry):
            zbuf[i // lanes, pl.ds((i % lanes) * 16, 16)] = zero16
            return carry

        lax.fori_loop(0, ZROWS * lanes, zstore, 0)

        def zcopy(k, carry):
            pltpu.sync_copy(zbuf, acc.at[pl.ds(sid * rows_per_tile + k * ZROWS, ZROWS)])
            return carry

        lax.fori_loop(0, nzcopy, zcopy, 0)
        plsc.subcore_barrier()

        def scatter_chunk(rbuf, ibuf):
            pltpu.sync_copy(rbuf, acc.at[ibuf], add=True)

        def body(j, carry):
            wait_load(rows_a, idx_a, sem_ra, sem_ia)
            start_load(2 * j + 1, rows_b, idx_b, sem_rb, sem_ib)
            scatter_chunk(rows_a, idx_a)
            wait_load(rows_b, idx_b, sem_rb, sem_ib)
            start_load(jnp.minimum(2 * j + 2, nload - 1), rows_a, idx_a, sem_ra, sem_ia)
            scatter_chunk(rows_b, idx_b)
            return carry

        lax.fori_loop(0, npair, body, 0)
        wait_load(rows_a, idx_a, sem_ra, sem_ia)

        if tail_chunks:
            @pl.when(wid < tail_chunks)
            def _tail():
                base = NW * epw + wid * CHUNK
                pltpu.sync_copy(es_hbm.at[pl.ds(base, CHUNK)], rows_a)
                pltpu.sync_copy(dst_hbm.at[pl.ds(base, CHUNK)], idx_a)
                pltpu.sync_copy(rows_a, acc.at[idx_a], add=True)

        plsc.subcore_barrier()

        @pl.when(sid < NS - 1)
        def _full():
            pltpu.sync_copy(
                acc.at[pl.ds(sid * rows_per_tile, rows_per_tile)],
                out_hbm.at[cid, pl.ds(sid * rows_per_tile, rows_per_tile)],
            )

        @pl.when(sid == NS - 1)
        def _last():
            pltpu.sync_copy(
                acc.at[pl.ds((NS - 1) * rows_per_tile, last_rows)],
                out_hbm.at[cid, pl.ds((NS - 1) * rows_per_tile, last_rows)],
            )

    return seg_sum(edge_state, edge_dst)


def _dense_body(nf_ref, p_ref, w_ref, b_ref, o_ref):
    d = nf_ref.shape[1]
    ns = p_ref[0] + p_ref[1]
    x = jnp.dot(nf_ref[...], w_ref[0:d, :], preferred_element_type=jnp.float32)
    y = jnp.dot(ns, w_ref[d:, :], preferred_element_type=jnp.float32)
    o_ref[...] = jnp.maximum(x + y + b_ref[...], 0.0)


def kernel(node_feature, edge_state, edge_dst, W, b):
    n_nodes, d = node_feature.shape
    units = W.shape[1]
    g = NS * ZROWS
    n_pad = ((n_nodes + g - 1) // g) * g
    partials = _segment_sum_sc(edge_state, edge_dst.astype(jnp.int32), n_nodes, n_pad)

    blk = 2000
    assert n_nodes % blk == 0
    grid = (n_nodes // blk,)
    out = pl.pallas_call(
        _dense_body,
        grid=grid,
        in_specs=[
            pl.BlockSpec((blk, d), lambda i: (i, 0)),
            pl.BlockSpec((NC, blk, d), lambda i: (0, i, 0)),
            pl.BlockSpec(W.shape, lambda i: (0, 0)),
            pl.BlockSpec((1, units), lambda i: (0, 0)),
        ],
        out_specs=pl.BlockSpec((blk, units), lambda i: (i, 0)),
        out_shape=jax.ShapeDtypeStruct((n_nodes, units), jnp.float32),
    )(node_feature, partials, W, b.reshape(1, units))
    return out


# speedup vs baseline: 1.2026x; 1.0447x over previous
import functools

import jax
import jax.numpy as jnp
from jax import lax
from jax.experimental import pallas as pl
from jax.experimental.pallas import tpu as pltpu
from jax.experimental.pallas import tpu_sc as plsc

NC = 2
NS = 16
NW = NC * NS

CHUNK = 160
IROWS = 2
ZROWS = 32


def _segment_sum_sc(edge_state, edge_dst, n_nodes, n_pad):
    n_edges, d = edge_state.shape
    nload = n_edges // (NW * CHUNK)
    assert nload % 2 == 0 and nload >= 4
    npair = nload // 2
    epw = nload * CHUNK
    ntail = n_edges - NW * epw
    assert ntail % CHUNK == 0
    tail_chunks = ntail // CHUNK
    assert tail_chunks <= NW
    rows_per_tile = n_pad // NS
    assert rows_per_tile * NS == n_pad and rows_per_tile % 8 == 0
    assert rows_per_tile % ZROWS == 0
    nzcopy = rows_per_tile // ZROWS
    last_rows = n_nodes - (NS - 1) * rows_per_tile
    assert 0 < last_rows <= rows_per_tile and last_rows % 8 == 0

    mesh = plsc.VectorSubcoreMesh(core_axis_name="c", subcore_axis_name="s")

    @functools.partial(
        pl.kernel,
        out_type=jax.ShapeDtypeStruct((NC, n_nodes, d), jnp.float32),
        mesh=mesh,
        scratch_types=[
            pltpu.VMEM_SHARED((n_pad, d), jnp.float32),
            pltpu.VMEM((CHUNK, d), jnp.float32),
            pltpu.VMEM((CHUNK, d), jnp.float32),
            pltpu.VMEM((CHUNK,), jnp.int32),
            pltpu.VMEM((CHUNK,), jnp.int32),
            pltpu.VMEM((ZROWS, d), jnp.float32),
            pltpu.SemaphoreType.DMA,
            pltpu.SemaphoreType.DMA,
            pltpu.SemaphoreType.DMA,
            pltpu.SemaphoreType.DMA,
        ],
    )
    def seg_sum(es_hbm, dst_hbm, out_hbm, acc, rows_a, rows_b, idx_a, idx_b,
                zbuf, sem_ra, sem_rb, sem_ia, sem_ib):
        cid = lax.axis_index("c")
        sid = lax.axis_index("s")
        wid = sid * NC + cid

        def start_load(li, rbuf, ibuf, rsem, isem):
            base = wid * epw + li * CHUNK
            pltpu.async_copy(es_hbm.at[pl.ds(base, CHUNK)], rbuf, rsem)
            pltpu.async_copy(dst_hbm.at[pl.ds(base, CHUNK)], ibuf, isem)

        def wait_load(rbuf, ibuf, rsem, isem):
            pltpu.make_async_copy(es_hbm.at[pl.ds(0, CHUNK)], rbuf, rsem).wait()
            pltpu.make_async_copy(dst_hbm.at[pl.ds(0, CHUNK)], ibuf, isem).wait()

        start_load(0, rows_a, idx_a, sem_ra, sem_ia)

        zero16 = jnp.zeros((16,), jnp.float32)
        lanes = d // 16

        def zstore(i, carry):
            zbuf[i // lanes, pl.ds((i % lanes) * 16, 16)] = zero16
            return carry

        lax.fori_loop(0, ZROWS * lanes, zstore, 0)

        def zcopy(k, carry):
            pltpu.sync_copy(zbuf, acc.at[pl.ds(sid * rows_per_tile + k * ZROWS, ZROWS)])
            return carry

        lax.fori_loop(0, nzcopy, zcopy, 0)
        plsc.subcore_barrier()

        def scatter_chunk(rbuf, ibuf):
            pltpu.sync_copy(rbuf, acc.at[ibuf], add=True)

        def body(j, carry):
            wait_load(rows_a, idx_a, sem_ra, sem_ia)
            start_load(2 * j + 1, rows_b, idx_b, sem_rb, sem_ib)
            scatter_chunk(rows_a, idx_a)
            wait_load(rows_b, idx_b, sem_rb, sem_ib)
            start_load(jnp.minimum(2 * j + 2, nload - 1), rows_a, idx_a, sem_ra, sem_ia)
            scatter_chunk(rows_b, idx_b)
            return carry

        lax.fori_loop(0, npair, body, 0)
        wait_load(rows_a, idx_a, sem_ra, sem_ia)

        if tail_chunks:
            @pl.when(wid < tail_chunks)
            def _tail():
                base = NW * epw + wid * CHUNK
                pltpu.sync_copy(es_hbm.at[pl.ds(base, CHUNK)], rows_a)
                pltpu.sync_copy(dst_hbm.at[pl.ds(base, CHUNK)], idx_a)
                pltpu.sync_copy(rows_a, acc.at[idx_a], add=True)

        plsc.subcore_barrier()

        @pl.when(sid < NS - 1)
        def _full():
            pltpu.sync_copy(
                acc.at[pl.ds(sid * rows_per_tile, rows_per_tile)],
                out_hbm.at[cid, pl.ds(sid * rows_per_tile, rows_per_tile)],
            )

        @pl.when(sid == NS - 1)
        def _last():
            pltpu.sync_copy(
                acc.at[pl.ds((NS - 1) * rows_per_tile, last_rows)],
                out_hbm.at[cid, pl.ds((NS - 1) * rows_per_tile, last_rows)],
            )

    return seg_sum(edge_state, edge_dst)


def _dense_body(nf_ref, p_ref, w_ref, b_ref, o_ref):
    d = nf_ref.shape[1]
    ns = p_ref[0] + p_ref[1]
    x = jnp.dot(nf_ref[...], w_ref[0:d, :], preferred_element_type=jnp.float32)
    y = jnp.dot(ns, w_ref[d:, :], preferred_element_type=jnp.float32)
    o_ref[...] = jnp.maximum(x + y + b_ref[...], 0.0)


def kernel(node_feature, edge_state, edge_dst, W, b):
    n_nodes, d = node_feature.shape
    units = W.shape[1]
    g = NS * ZROWS
    n_pad = ((n_nodes + g - 1) // g) * g
    partials = _segment_sum_sc(edge_state, edge_dst.astype(jnp.int32), n_nodes, n_pad)

    blk = 2000
    assert n_nodes % blk == 0
    grid = (n_nodes // blk,)
    out = pl.pallas_call(
        _dense_body,
        grid=grid,
        in_specs=[
            pl.BlockSpec((blk, d), lambda i: (i, 0)),
            pl.BlockSpec((NC, blk, d), lambda i: (0, i, 0)),
            pl.BlockSpec(W.shape, lambda i: (0, 0)),
            pl.BlockSpec((1, units), lambda i: (0, 0)),
        ],
        out_specs=pl.BlockSpec((blk, units), lambda i: (i, 0)),
        out_shape=jax.ShapeDtypeStruct((n_nodes, units), jnp.float32),
    )(node_feature, partials, W, b.reshape(1, units))
    return out


# speedup vs baseline: 1.2069x; 1.0035x over previous
import functools

import jax
import jax.numpy as jnp
from jax import lax
from jax.experimental import pallas as pl
from jax.experimental.pallas import tpu as pltpu
from jax.experimental.pallas import tpu_sc as plsc

NC = 2
NS = 16
NW = NC * NS

CHUNK = 160
IROWS = 2
ZROWS = 32


def _segment_sum_sc(edge_state, edge_dst, n_nodes, n_pad):
    n_edges, d = edge_state.shape
    nload = n_edges // (NW * CHUNK)
    assert nload % 2 == 0 and nload >= 4
    npair = nload // 2
    epw = nload * CHUNK
    ntail = n_edges - NW * epw
    assert ntail % CHUNK == 0
    tail_chunks = ntail // CHUNK
    assert tail_chunks <= NW
    rows_per_tile = n_pad // NS
    assert rows_per_tile * NS == n_pad and rows_per_tile % 8 == 0
    assert rows_per_tile % ZROWS == 0
    nzcopy = rows_per_tile // ZROWS
    last_rows = n_nodes - (NS - 1) * rows_per_tile
    assert 0 < last_rows <= rows_per_tile and last_rows % 8 == 0

    mesh = plsc.VectorSubcoreMesh(core_axis_name="c", subcore_axis_name="s")

    @functools.partial(
        pl.kernel,
        out_type=jax.ShapeDtypeStruct((NC, n_nodes, d), jnp.float32),
        mesh=mesh,
        scratch_types=[
            pltpu.VMEM_SHARED((n_pad, d), jnp.float32),
            pltpu.VMEM((CHUNK, d), jnp.float32),
            pltpu.VMEM((CHUNK, d), jnp.float32),
            pltpu.VMEM((CHUNK,), jnp.int32),
            pltpu.VMEM((CHUNK,), jnp.int32),
            pltpu.VMEM((ZROWS, d), jnp.float32),
            pltpu.SemaphoreType.DMA,
            pltpu.SemaphoreType.DMA,
            pltpu.SemaphoreType.DMA,
            pltpu.SemaphoreType.DMA,
        ],
    )
    def seg_sum(es_hbm, dst_hbm, out_hbm, acc, rows_a, rows_b, idx_a, idx_b,
                zbuf, sem_ra, sem_rb, sem_ia, sem_ib):
        cid = lax.axis_index("c")
        sid = lax.axis_index("s")
        wid = sid * NC + cid

        def start_load(li, rbuf, ibuf, rsem, isem):
            base = wid * epw + li * CHUNK
            pltpu.async_copy(es_hbm.at[pl.ds(base, CHUNK)], rbuf, rsem)
            pltpu.async_copy(dst_hbm.at[pl.ds(base, CHUNK)], ibuf, isem)

        def wait_load(rbuf, ibuf, rsem, isem):
            pltpu.make_async_copy(es_hbm.at[pl.ds(0, CHUNK)], rbuf, rsem).wait()
            pltpu.make_async_copy(dst_hbm.at[pl.ds(0, CHUNK)], ibuf, isem).wait()

        start_load(0, rows_a, idx_a, sem_ra, sem_ia)

        zero16 = jnp.zeros((16,), jnp.float32)
        lanes = d // 16

        def zstore(i, carry):
            zbuf[i // lanes, pl.ds((i % lanes) * 16, 16)] = zero16
            return carry

        lax.fori_loop(0, ZROWS * lanes, zstore, 0)

        def zcopy(k, carry):
            pltpu.sync_copy(zbuf, acc.at[pl.ds(sid * rows_per_tile + k * ZROWS, ZROWS)])
            return carry

        lax.fori_loop(0, nzcopy, zcopy, 0)
        plsc.subcore_barrier()

        def scatter_chunk(rbuf, ibuf):
            pltpu.sync_copy(rbuf, acc.at[ibuf], add=True)

        def body(j, carry):
            wait_load(rows_a, idx_a, sem_ra, sem_ia)
            start_load(2 * j + 1, rows_b, idx_b, sem_rb, sem_ib)
            scatter_chunk(rows_a, idx_a)
            wait_load(rows_b, idx_b, sem_rb, sem_ib)
            start_load(jnp.minimum(2 * j + 2, nload - 1), rows_a, idx_a, sem_ra, sem_ia)
            scatter_chunk(rows_b, idx_b)
            return carry

        lax.fori_loop(0, npair, body, 0)
        wait_load(rows_a, idx_a, sem_ra, sem_ia)

        if tail_chunks:
            @pl.when(wid < tail_chunks)
            def _tail():
                base = NW * epw + wid * CHUNK
                pltpu.sync_copy(es_hbm.at[pl.ds(base, CHUNK)], rows_a)
                pltpu.sync_copy(dst_hbm.at[pl.ds(base, CHUNK)], idx_a)
                pltpu.sync_copy(rows_a, acc.at[idx_a], add=True)

        plsc.subcore_barrier()

        @pl.when(sid < NS - 1)
        def _full():
            pltpu.sync_copy(
                acc.at[pl.ds(sid * rows_per_tile, rows_per_tile)],
                out_hbm.at[cid, pl.ds(sid * rows_per_tile, rows_per_tile)],
            )

        @pl.when(sid == NS - 1)
        def _last():
            pltpu.sync_copy(
                acc.at[pl.ds((NS - 1) * rows_per_tile, last_rows)],
                out_hbm.at[cid, pl.ds((NS - 1) * rows_per_tile, last_rows)],
            )

    return seg_sum(edge_state, edge_dst)


def _dense_body(nf_ref, p_ref, w_ref, b_ref, o_ref):
    d = nf_ref.shape[1]
    ns = p_ref[0] + p_ref[1]
    x = jnp.dot(nf_ref[...], w_ref[0:d, :], preferred_element_type=jnp.float32)
    y = jnp.dot(ns, w_ref[d:, :], preferred_element_type=jnp.float32)
    o_ref[...] = jnp.maximum(x + y + b_ref[...], 0.0)


def kernel(node_feature, edge_state, edge_dst, W, b):
    n_nodes, d = node_feature.shape
    units = W.shape[1]
    g = NS * ZROWS
    n_pad = ((n_nodes + g - 1) // g) * g
    partials = _segment_sum_sc(edge_state, edge_dst.astype(jnp.int32), n_nodes, n_pad)

    blk = 5000
    assert n_nodes % blk == 0
    grid = (n_nodes // blk,)
    out = pl.pallas_call(
        _dense_body,
        grid=grid,
        in_specs=[
            pl.BlockSpec((blk, d), lambda i: (i, 0)),
            pl.BlockSpec((NC, blk, d), lambda i: (0, i, 0)),
            pl.BlockSpec(W.shape, lambda i: (0, 0)),
            pl.BlockSpec((1, units), lambda i: (0, 0)),
        ],
        out_specs=pl.BlockSpec((blk, units), lambda i: (i, 0)),
        out_shape=jax.ShapeDtypeStruct((n_nodes, units), jnp.float32),
    )(node_feature, partials, W, b.reshape(1, units))
    return out


# speedup vs baseline: 1.2295x; 1.0187x over previous
import functools

import jax
import jax.numpy as jnp
from jax import lax
from jax.experimental import pallas as pl
from jax.experimental.pallas import tpu as pltpu
from jax.experimental.pallas import tpu_sc as plsc

NC = 2
NS = 16
NW = NC * NS

CHUNK = 176
ZROWS = 16


def _segment_sum_sc(edge_state, edge_dst, n_nodes, n_pad):
    n_edges, d = edge_state.shape
    nload = n_edges // (NW * CHUNK)
    assert nload % 2 == 0 and nload >= 4
    npair = nload // 2
    epw = nload * CHUNK
    ntail = n_edges - NW * epw
    tail_per_w = ntail // NW
    assert tail_per_w * NW == ntail
    assert tail_per_w % 8 == 0 and tail_per_w <= CHUNK
    rows_per_tile = n_pad // NS
    assert rows_per_tile * NS == n_pad and rows_per_tile % 8 == 0
    assert rows_per_tile % ZROWS == 0
    nzcopy = rows_per_tile // ZROWS
    last_rows = n_nodes - (NS - 1) * rows_per_tile
    assert 0 < last_rows <= rows_per_tile and last_rows % 8 == 0

    mesh = plsc.VectorSubcoreMesh(core_axis_name="c", subcore_axis_name="s")

    @functools.partial(
        pl.kernel,
        out_type=jax.ShapeDtypeStruct((NC, n_nodes, d), jnp.float32),
        mesh=mesh,
        scratch_types=[
            pltpu.VMEM_SHARED((n_pad, d), jnp.float32),
            pltpu.VMEM((CHUNK, d), jnp.float32),
            pltpu.VMEM((CHUNK, d), jnp.float32),
            pltpu.VMEM((CHUNK,), jnp.int32),
            pltpu.VMEM((CHUNK,), jnp.int32),
            pltpu.VMEM((max(tail_per_w, 8),), jnp.int32),
            pltpu.VMEM((ZROWS, d), jnp.float32),
            pltpu.SemaphoreType.DMA,
            pltpu.SemaphoreType.DMA,
            pltpu.SemaphoreType.DMA,
            pltpu.SemaphoreType.DMA,
        ],
    )
    def seg_sum(es_hbm, dst_hbm, out_hbm, acc, rows_a, rows_b, idx_a, idx_b,
                idx_t, zbuf, sem_ra, sem_rb, sem_ia, sem_ib):
        cid = lax.axis_index("c")
        sid = lax.axis_index("s")
        wid = sid * NC + cid

        def start_load(li, rbuf, ibuf, rsem, isem):
            base = wid * epw + li * CHUNK
            pltpu.async_copy(es_hbm.at[pl.ds(base, CHUNK)], rbuf, rsem)
            pltpu.async_copy(dst_hbm.at[pl.ds(base, CHUNK)], ibuf, isem)

        def wait_load(rbuf, ibuf, rsem, isem):
            pltpu.make_async_copy(es_hbm.at[pl.ds(0, CHUNK)], rbuf, rsem).wait()
            pltpu.make_async_copy(dst_hbm.at[pl.ds(0, CHUNK)], ibuf, isem).wait()

        start_load(0, rows_a, idx_a, sem_ra, sem_ia)

        zero16 = jnp.zeros((16,), jnp.float32)
        lanes = d // 16

        def zstore(i, carry):
            zbuf[i // lanes, pl.ds((i % lanes) * 16, 16)] = zero16
            return carry

        lax.fori_loop(0, ZROWS * lanes, zstore, 0)

        def zcopy(k, carry):
            pltpu.sync_copy(zbuf, acc.at[pl.ds(sid * rows_per_tile + k * ZROWS, ZROWS)])
            return carry

        lax.fori_loop(0, nzcopy, zcopy, 0)
        plsc.subcore_barrier()

        def scatter_chunk(rbuf, ibuf):
            pltpu.sync_copy(rbuf, acc.at[ibuf], add=True)

        def body(j, carry):
            wait_load(rows_a, idx_a, sem_ra, sem_ia)
            start_load(2 * j + 1, rows_b, idx_b, sem_rb, sem_ib)
            scatter_chunk(rows_a, idx_a)
            wait_load(rows_b, idx_b, sem_rb, sem_ib)
            start_load(jnp.minimum(2 * j + 2, nload - 1), rows_a, idx_a, sem_ra, sem_ia)
            scatter_chunk(rows_b, idx_b)
            return carry

        lax.fori_loop(0, npair, body, 0)
        wait_load(rows_a, idx_a, sem_ra, sem_ia)

        if tail_per_w:
            base = NW * epw + wid * tail_per_w
            pltpu.sync_copy(es_hbm.at[pl.ds(base, tail_per_w)],
                            rows_a.at[pl.ds(0, tail_per_w)])
            pltpu.sync_copy(dst_hbm.at[pl.ds(base, tail_per_w)], idx_t)
            pltpu.sync_copy(rows_a.at[pl.ds(0, tail_per_w)], acc.at[idx_t], add=True)

        plsc.subcore_barrier()

        @pl.when(sid < NS - 1)
        def _full():
            pltpu.sync_copy(
                acc.at[pl.ds(sid * rows_per_tile, rows_per_tile)],
                out_hbm.at[cid, pl.ds(sid * rows_per_tile, rows_per_tile)],
            )

        @pl.when(sid == NS - 1)
        def _last():
            pltpu.sync_copy(
                acc.at[pl.ds((NS - 1) * rows_per_tile, last_rows)],
                out_hbm.at[cid, pl.ds((NS - 1) * rows_per_tile, last_rows)],
            )

    return seg_sum(edge_state, edge_dst)


def _dense_body(nf_ref, p_ref, w_ref, b_ref, o_ref):
    d = nf_ref.shape[1]
    ns = p_ref[0] + p_ref[1]
    x = jnp.dot(nf_ref[...], w_ref[0:d, :], preferred_element_type=jnp.float32)
    y = jnp.dot(ns, w_ref[d:, :], preferred_element_type=jnp.float32)
    o_ref[...] = jnp.maximum(x + y + b_ref[...], 0.0)


def kernel(node_feature, edge_state, edge_dst, W, b):
    n_nodes, d = node_feature.shape
    units = W.shape[1]
    g = NS * ZROWS
    n_pad = ((n_nodes + g - 1) // g) * g
    partials = _segment_sum_sc(edge_state, edge_dst.astype(jnp.int32), n_nodes, n_pad)

    blk = 5000
    assert n_nodes % blk == 0
    grid = (n_nodes // blk,)
    out = pl.pallas_call(
        _dense_body,
        grid=grid,
        in_specs=[
            pl.BlockSpec((blk, d), lambda i: (i, 0)),
            pl.BlockSpec((NC, blk, d), lambda i: (0, i, 0)),
            pl.BlockSpec(W.shape, lambda i: (0, 0)),
            pl.BlockSpec((1, units), lambda i: (0, 0)),
        ],
        out_specs=pl.BlockSpec((blk, units), lambda i: (i, 0)),
        out_shape=jax.ShapeDtypeStruct((n_nodes, units), jnp.float32),
    )(node_feature, partials, W, b.reshape(1, units))
    return out
